# baseline (speedup 1.0000x reference)
import dataclasses

import jax
import jax.numpy as jnp
from jax import lax
from jax.experimental import pallas as pl
from jax.experimental.pallas import tpu as pltpu
from jax.experimental.pallas import tpu_sc as plsc

N = 10000
E = 320000
D = 128
H = 4
C = 32
NPAD = 10240
ROWB = 1280
CHUNK = 32
SUPER = 2
NSUPERS = 162
NCHUNKS = SUPER * NSUPERS
PER_TILE = NCHUNKS * CHUNK
EPAD = PER_TILE * 32
ROWS_PER_TILE = NPAD // 16
DROWS = NPAD // 8
DROWS_PER_TILE = DROWS // 16


def _lin_body(x_ref, wl_ref, bl_ref, wr_ref, br_ref, xl_ref, xr_ref):
    xb = x_ref[...]
    xl_ref[...] = jnp.dot(xb, wl_ref[...],
                          preferred_element_type=jnp.float32) + bl_ref[...]
    xr_ref[...] = jnp.dot(xb, wr_ref[...],
                          preferred_element_type=jnp.float32) + br_ref[...]


def _linear(x_pad, W_l, b_l, W_r, b_r):
    grid = NPAD // ROWB
    return pl.pallas_call(
        _lin_body,
        grid=(grid,),
        in_specs=[
            pl.BlockSpec((ROWB, D), lambda i: (i, 0)),
            pl.BlockSpec((D, D), lambda i: (0, 0)),
            pl.BlockSpec((1, D), lambda i: (0, 0)),
            pl.BlockSpec((D, D), lambda i: (0, 0)),
            pl.BlockSpec((1, D), lambda i: (0, 0)),
        ],
        out_specs=[
            pl.BlockSpec((ROWB, D), lambda i: (i, 0)),
            pl.BlockSpec((ROWB, D), lambda i: (i, 0)),
        ],
        out_shape=[
            jax.ShapeDtypeStruct((NPAD, D), jnp.float32),
            jax.ShapeDtypeStruct((NPAD, D), jnp.float32),
        ],
    )(x_pad, W_l, b_l.reshape(1, D), W_r, b_r.reshape(1, D))


def _edge_kernel(xl_hbm, xr_hbm, src_hbm, dst_hbm, att_hbm,
                 out_msg_hbm, out_den_hbm,
                 srcb0, srcb1, dstb0, dstb1, d8b0, d8b1,
                 rows_l0, rows_l1, rows_r0, rows_r1, att_v,
                 acc, acc_den,
                 sem_i0, sem_i1, sem_l0, sem_l1, sem_r0, sem_r1,
                 sem_sl0, sem_sl1, sem_sr0, sem_sr1):
    c_ax = lax.axis_index("c")
    s = lax.axis_index("s")
    wid = c_ax * 16 + s
    tile_row0 = wid * (PER_TILE // CHUNK)

    srcb = (srcb0, srcb1)
    dstb = (dstb0, dstb1)
    d8b = (d8b0, d8b1)
    rows_l = (rows_l0, rows_l1)
    rows_r = (rows_r0, rows_r1)
    sem_i = (sem_i0, sem_i1)
    sem_l = (sem_l0, sem_l1)
    sem_r = (sem_r0, sem_r1)
    sem_sl = (sem_sl0, sem_sl1)
    sem_sr = (sem_sr0, sem_sr1)

    pltpu.sync_copy(att_hbm, att_v)

    zeros16 = jnp.zeros((16,), jnp.float32)

    @pl.loop(0, CHUNK)
    def _(i):
        for k in range(D // 16):
            rows_l0[i, pl.ds(k * 16, 16)] = zeros16

    @pl.loop(0, ROWS_PER_TILE // CHUNK)
    def _(t):
        pltpu.sync_copy(rows_l0,
                        acc.at[pl.ds(s * ROWS_PER_TILE + t * CHUNK, CHUNK)])
    _rem = ROWS_PER_TILE % CHUNK
    if _rem:
        pltpu.sync_copy(
            rows_l0.at[pl.ds(0, _rem)],
            acc.at[pl.ds(s * ROWS_PER_TILE + ROWS_PER_TILE - _rem, _rem)])
    @pl.loop(0, DROWS_PER_TILE // CHUNK)
    def _(t):
        pltpu.sync_copy(
            rows_l0,
            acc_den.at[pl.ds(s * DROWS_PER_TILE + t * CHUNK, CHUNK)])
    _drem = DROWS_PER_TILE % CHUNK
    if _drem:
        pltpu.sync_copy(
            rows_l0.at[pl.ds(0, _drem)],
            acc_den.at[pl.ds(s * DROWS_PER_TILE + DROWS_PER_TILE - _drem,
                             _drem)])

    plsc.subcore_barrier()

    lane = lax.iota(jnp.int32, 16)
    onehots = [jnp.where(lane == h, 1.0, 0.0).astype(jnp.float32)
               for h in range(H)]

    def issue_idx(sup, b):
        row = tile_row0 + sup * SUPER
        pltpu.async_copy(src_hbm.at[pl.ds(row, SUPER)], srcb[b], sem_i[b])
        pltpu.async_copy(dst_hbm.at[pl.ds(row, SUPER)], dstb[b], sem_i[b])

    def wait_idx(b):
        pltpu.make_async_copy(src_hbm.at[pl.ds(0, SUPER)], srcb[b],
                              sem_i[b]).wait()
        pltpu.make_async_copy(dst_hbm.at[pl.ds(0, SUPER)], dstb[b],
                              sem_i[b]).wait()

    def fill_d8(b):
        for cc in range(SUPER):
            for k in range(CHUNK // 16):
                d8b[b][cc, pl.ds(k * 16, 16)] = lax.shift_right_logical(
                    dstb[b][cc, pl.ds(k * 16, 16)], 3)

    def issue_gather(ib, cc, rb):
        pltpu.async_copy(xl_hbm.at[srcb[ib].at[cc]], rows_l[rb], sem_l[rb])
        pltpu.async_copy(xr_hbm.at[dstb[ib].at[cc]], rows_r[rb], sem_r[rb])

    def wait_gather(rb):
        pltpu.make_async_copy(xl_hbm.at[srcb[0].at[0]], rows_l[rb],
                              sem_l[rb]).wait()
        pltpu.make_async_copy(xr_hbm.at[dstb[0].at[0]], rows_r[rb],
                              sem_r[rb]).wait()

    def compute_chunk(ib, cc, rb):
        rl = rows_l[rb]
        rr = rows_r[rb]

        @pl.loop(0, CHUNK)
        def _(i):
            lchunks = []
            tsum = []
            for k in range(8):
                zl = rl[i, pl.ds(k * 16, 16)]
                zr = rr[i, pl.ds(k * 16, 16)]
                z = zl + zr
                z = jnp.maximum(z, 0.2 * z)
                lchunks.append(zl)
                tsum.append(z * att_v[pl.ds(k * 16, 16)])
            exvs = []
            for h in range(H):
                lh = jnp.sum(tsum[2 * h] + tsum[2 * h + 1])
                exvs.append(jnp.exp(jnp.full((16,), lh, jnp.float32)))
            for k in range(8):
                rl[i, pl.ds(k * 16, 16)] = lchunks[k] * exvs[k // 2]
            den = (exvs[0] * onehots[0] + exvs[1] * onehots[1]
                   + exvs[2] * onehots[2] + exvs[3] * onehots[3])
            dsplat = plsc.load_gather(
                dstb[ib], [jnp.full((16,), cc, jnp.int32),
                           jnp.full((16,), i, jnp.int32)])
            dm8 = jnp.bitwise_and(dsplat, 7)
            for j in range(8):
                rr[i, pl.ds(j * 16, 16)] = jnp.where(dm8 == j, den, zeros16)

    def scatter_chunk(ib, cc, rb):
        pltpu.async_copy(rows_l[rb], acc.at[dstb[ib].at[cc]], sem_sl[rb],
                         add=True)
        pltpu.async_copy(rows_r[rb], acc_den.at[d8b[ib].at[cc]], sem_sr[rb],
                         add=True)

    def wait_scatter(rb):
        pltpu.make_async_copy(rows_l[rb], acc.at[dstb[0].at[0]],
                              sem_sl[rb]).wait()
        pltpu.make_async_copy(rows_r[rb], acc_den.at[d8b[0].at[0]],
                              sem_sr[rb]).wait()

    issue_idx(0, 0)
    issue_idx(1, 1)
    wait_idx(0)
    fill_d8(0)
    issue_gather(0, 0, 0)

    @pl.loop(0, NSUPERS // 2)
    def _(spair):
        for sb in range(2):
            sup = 2 * spair + sb
            for cc in range(SUPER):
                rb = cc % 2
                wait_gather(rb)
                if cc < SUPER - 1:
                    if cc == 0 and sb == 0:
                        @pl.when(spair > 0)
                        def _():
                            wait_scatter(1 - rb)
                    elif cc == 0:
                        wait_scatter(1 - rb)
                    else:
                        wait_scatter(1 - rb)
                    issue_gather(sb, cc + 1, 1 - rb)
                elif sb == 0:
                    wait_idx(1)
                    fill_d8(1)
                    wait_scatter(1 - rb)
                    issue_gather(1, 0, 1 - rb)
                else:
                    @pl.when(spair < NSUPERS // 2 - 1)
                    def _():
                        wait_idx(0)
                        fill_d8(0)
                        wait_scatter(1 - rb)
                        issue_gather(0, 0, 1 - rb)
                compute_chunk(sb, cc, rb)
                scatter_chunk(sb, cc, rb)
            @pl.when(spair < NSUPERS // 2 - 1)
            def _():
                issue_idx(sup + 2, sb)

    wait_scatter(0)
    wait_scatter(1)

    plsc.subcore_barrier()

    pltpu.sync_copy(acc.at[pl.ds(s * ROWS_PER_TILE, ROWS_PER_TILE)],
                    out_msg_hbm.at[c_ax,
                                   pl.ds(s * ROWS_PER_TILE, ROWS_PER_TILE)])
    pltpu.sync_copy(
        acc_den.at[pl.ds(s * DROWS_PER_TILE, DROWS_PER_TILE)],
        out_den_hbm.at[c_ax, pl.ds(s * DROWS_PER_TILE, DROWS_PER_TILE)])


def _edge_pass(xl_pad, xr_pad, src_all, dst_all, att_flat):
    mesh = plsc.VectorSubcoreMesh(core_axis_name="c", subcore_axis_name="s")
    cp = pltpu.CompilerParams()
    if "needs_layout_passes" in pltpu.CompilerParams.__dataclass_fields__:
        cp = dataclasses.replace(cp, needs_layout_passes=False)
    run = pl.kernel(
        _edge_kernel,
        mesh=mesh,
        compiler_params=cp,
        out_type=[
            jax.ShapeDtypeStruct((2, NPAD, D), jnp.float32),
            jax.ShapeDtypeStruct((2, DROWS, D), jnp.float32),
        ],
        scratch_types=[
            pltpu.VMEM((SUPER, CHUNK), jnp.int32),
            pltpu.VMEM((SUPER, CHUNK), jnp.int32),
            pltpu.VMEM((SUPER, CHUNK), jnp.int32),
            pltpu.VMEM((SUPER, CHUNK), jnp.int32),
            pltpu.VMEM((SUPER, CHUNK), jnp.int32),
            pltpu.VMEM((SUPER, CHUNK), jnp.int32),
            pltpu.VMEM((CHUNK, D), jnp.float32),
            pltpu.VMEM((CHUNK, D), jnp.float32),
            pltpu.VMEM((CHUNK, D), jnp.float32),
            pltpu.VMEM((CHUNK, D), jnp.float32),
            pltpu.VMEM((D,), jnp.float32),
            pltpu.VMEM_SHARED((NPAD, D), jnp.float32),
            pltpu.VMEM_SHARED((DROWS, D), jnp.float32),
            pltpu.SemaphoreType.DMA,
            pltpu.SemaphoreType.DMA,
            pltpu.SemaphoreType.DMA,
            pltpu.SemaphoreType.DMA,
            pltpu.SemaphoreType.DMA,
            pltpu.SemaphoreType.DMA,
            pltpu.SemaphoreType.DMA,
            pltpu.SemaphoreType.DMA,
            pltpu.SemaphoreType.DMA,
            pltpu.SemaphoreType.DMA,
        ],
    )
    return run(xl_pad, xr_pad, src_all, dst_all, att_flat)


def _epi_body(num_ref, den_ref, x_ref, bias_ref, gamma_ref, beta_ref, o_ref):
    nv = num_ref[...]
    num = nv[0] + nv[1]
    dv = den_ref[...]
    den = dv[0] + dv[1]
    row = lax.broadcasted_iota(jnp.int32, (16, D), 0)
    col = lax.broadcasted_iota(jnp.int32, (16, D), 1)
    sel = jnp.where(col // C == row, 1.0, 0.0).astype(jnp.float32)
    denf = jax.lax.dot_general(den, sel, (((1,), (0,)), ((), ())),
                               preferred_element_type=jnp.float32)
    y = num / (denf + 1e-16) + bias_ref[...] + x_ref[...]
    mu = jnp.mean(y, axis=1, keepdims=True)
    d = y - mu
    var = jnp.mean(d * d, axis=1, keepdims=True)
    o_ref[...] = d * lax.rsqrt(var + 1e-5) * gamma_ref[...] + beta_ref[...]


def _epilogue(numer, den, x_pad, bias, gamma, beta):
    grid = NPAD // ROWB
    return pl.pallas_call(
        _epi_body,
        grid=(grid,),
        in_specs=[
            pl.BlockSpec((2, ROWB, D), lambda i: (0, i, 0)),
            pl.BlockSpec((2, ROWB, 16), lambda i: (0, i, 0)),
            pl.BlockSpec((ROWB, D), lambda i: (i, 0)),
            pl.BlockSpec((1, D), lambda i: (0, 0)),
            pl.BlockSpec((1, D), lambda i: (0, 0)),
            pl.BlockSpec((1, D), lambda i: (0, 0)),
        ],
        out_specs=pl.BlockSpec((ROWB, D), lambda i: (i, 0)),
        out_shape=jax.ShapeDtypeStruct((NPAD, D), jnp.float32),
    )(numer, den, x_pad, bias.reshape(1, D), gamma.reshape(1, D),
      beta.reshape(1, D))


def kernel(x, edge_index, W_l, b_l, W_r, b_r, att, bias, gamma, beta):
    x_pad = jnp.pad(x, ((0, NPAD - N), (0, 0)))
    loop = jnp.arange(N, dtype=jnp.int32)
    pad_idx = jnp.full((EPAD - E - N,), N, dtype=jnp.int32)
    src_all = jnp.concatenate([edge_index[0], loop, pad_idx])
    dst_all = jnp.concatenate([edge_index[1], loop, pad_idx])
    src_2d = src_all.reshape(EPAD // CHUNK, CHUNK)
    dst_2d = dst_all.reshape(EPAD // CHUNK, CHUNK)
    att_flat = att.reshape(H * C)

    xl_pad, xr_pad = _linear(x_pad, W_l, b_l, W_r, b_r)
    numer, den_parts = _edge_pass(xl_pad, xr_pad, src_2d, dst_2d, att_flat)
    den = den_parts.reshape(2, NPAD, 16)
    out = _epilogue(numer, den, x_pad, bias, gamma, beta)
    return out[:N]


# speedup vs baseline: 1.0074x; 1.0074x over previous
import dataclasses

import jax
import jax.numpy as jnp
from jax import lax
from jax.experimental import pallas as pl
from jax.experimental.pallas import tpu as pltpu
from jax.experimental.pallas import tpu_sc as plsc

N = 10000
E = 320000
D = 128
H = 4
C = 32
NPAD = 10240
ROWB = 1280
CHUNK = 48
SUPER = 2
NSUPERS = 108
NCHUNKS = SUPER * NSUPERS
PER_TILE = NCHUNKS * CHUNK
EPAD = PER_TILE * 32
ROWS_PER_TILE = NPAD // 16
DROWS = NPAD // 8
DROWS_PER_TILE = DROWS // 16


def _lin_body(x_ref, wl_ref, bl_ref, wr_ref, br_ref, xl_ref, xr_ref):
    xb = x_ref[...]
    xl_ref[...] = jnp.dot(xb, wl_ref[...],
                          preferred_element_type=jnp.float32) + bl_ref[...]
    xr_ref[...] = jnp.dot(xb, wr_ref[...],
                          preferred_element_type=jnp.float32) + br_ref[...]


def _linear(x_pad, W_l, b_l, W_r, b_r):
    grid = NPAD // ROWB
    return pl.pallas_call(
        _lin_body,
        grid=(grid,),
        in_specs=[
            pl.BlockSpec((ROWB, D), lambda i: (i, 0)),
            pl.BlockSpec((D, D), lambda i: (0, 0)),
            pl.BlockSpec((1, D), lambda i: (0, 0)),
            pl.BlockSpec((D, D), lambda i: (0, 0)),
            pl.BlockSpec((1, D), lambda i: (0, 0)),
        ],
        out_specs=[
            pl.BlockSpec((ROWB, D), lambda i: (i, 0)),
            pl.BlockSpec((ROWB, D), lambda i: (i, 0)),
        ],
        out_shape=[
            jax.ShapeDtypeStruct((NPAD, D), jnp.float32),
            jax.ShapeDtypeStruct((NPAD, D), jnp.float32),
        ],
    )(x_pad, W_l, b_l.reshape(1, D), W_r, b_r.reshape(1, D))


def _edge_kernel(xl_hbm, xr_hbm, src_hbm, dst_hbm, att_hbm,
                 out_msg_hbm, out_den_hbm,
                 srcb0, srcb1, dstb0, dstb1, d8b0, d8b1,
                 rows_l0, rows_l1, rows_r0, rows_r1, att_v,
                 acc, acc_den,
                 sem_i0, sem_i1, sem_l0, sem_l1, sem_r0, sem_r1,
                 sem_sl0, sem_sl1, sem_sr0, sem_sr1):
    c_ax = lax.axis_index("c")
    s = lax.axis_index("s")
    wid = c_ax * 16 + s
    tile_row0 = wid * NCHUNKS

    srcb = (srcb0, srcb1)
    dstb = (dstb0, dstb1)
    d8b = (d8b0, d8b1)
    rows_l = (rows_l0, rows_l1)
    rows_r = (rows_r0, rows_r1)
    sem_i = (sem_i0, sem_i1)
    sem_l = (sem_l0, sem_l1)
    sem_r = (sem_r0, sem_r1)
    sem_sl = (sem_sl0, sem_sl1)
    sem_sr = (sem_sr0, sem_sr1)

    pltpu.sync_copy(att_hbm, att_v)

    zeros16 = jnp.zeros((16,), jnp.float32)

    @pl.loop(0, CHUNK)
    def _(i):
        for k in range(D // 16):
            rows_l0[i, pl.ds(k * 16, 16)] = zeros16

    @pl.loop(0, ROWS_PER_TILE // CHUNK)
    def _(t):
        pltpu.sync_copy(rows_l0,
                        acc.at[pl.ds(s * ROWS_PER_TILE + t * CHUNK, CHUNK)])
    _rem = ROWS_PER_TILE % CHUNK
    if _rem:
        pltpu.sync_copy(
            rows_l0.at[pl.ds(0, _rem)],
            acc.at[pl.ds(s * ROWS_PER_TILE + ROWS_PER_TILE - _rem, _rem)])
    pltpu.sync_copy(rows_l0, acc_den.at[pl.ds(s * DROWS_PER_TILE, CHUNK)])
    pltpu.sync_copy(rows_l0.at[pl.ds(0, DROWS_PER_TILE - CHUNK)],
                    acc_den.at[pl.ds(s * DROWS_PER_TILE + CHUNK,
                                     DROWS_PER_TILE - CHUNK)])

    plsc.subcore_barrier()

    lane = lax.iota(jnp.int32, 16)
    onehots = [jnp.where(lane == h, 1.0, 0.0).astype(jnp.float32)
               for h in range(H)]

    def issue_idx(sup, b):
        row = tile_row0 + sup * SUPER
        pltpu.async_copy(src_hbm.at[pl.ds(row, SUPER)], srcb[b], sem_i[b])
        pltpu.async_copy(dst_hbm.at[pl.ds(row, SUPER)], dstb[b], sem_i[b])

    def wait_idx(b):
        pltpu.make_async_copy(src_hbm.at[pl.ds(0, SUPER)], srcb[b],
                              sem_i[b]).wait()
        pltpu.make_async_copy(dst_hbm.at[pl.ds(0, SUPER)], dstb[b],
                              sem_i[b]).wait()

    def fill_d8(b):
        for cc in range(SUPER):
            for k in range(CHUNK // 16):
                d8b[b][cc, pl.ds(k * 16, 16)] = lax.shift_right_logical(
                    dstb[b][cc, pl.ds(k * 16, 16)], 3)

    def issue_gather(ib, cc, rb):
        pltpu.async_copy(xl_hbm.at[srcb[ib].at[cc]], rows_l[rb], sem_l[rb])
        pltpu.async_copy(xr_hbm.at[dstb[ib].at[cc]], rows_r[rb], sem_r[rb])

    def wait_gather(rb):
        pltpu.make_async_copy(xl_hbm.at[srcb[0].at[0]], rows_l[rb],
                              sem_l[rb]).wait()
        pltpu.make_async_copy(xr_hbm.at[dstb[0].at[0]], rows_r[rb],
                              sem_r[rb]).wait()

    def compute_chunk(ib, cc, rb):
        rl = rows_l[rb]
        rr = rows_r[rb]

        @pl.loop(0, CHUNK)
        def _(i):
            lchunks = []
            tsum = []
            for k in range(8):
                zl = rl[i, pl.ds(k * 16, 16)]
                zr = rr[i, pl.ds(k * 16, 16)]
                z = zl + zr
                z = jnp.maximum(z, 0.2 * z)
                lchunks.append(zl)
                tsum.append(z * att_v[pl.ds(k * 16, 16)])
            exvs = []
            for h in range(H):
                lh = jnp.sum(tsum[2 * h] + tsum[2 * h + 1])
                exvs.append(jnp.exp(jnp.full((16,), lh, jnp.float32)))
            for k in range(8):
                rl[i, pl.ds(k * 16, 16)] = lchunks[k] * exvs[k // 2]
            den = (exvs[0] * onehots[0] + exvs[1] * onehots[1]
                   + exvs[2] * onehots[2] + exvs[3] * onehots[3])
            dsplat = plsc.load_gather(
                dstb[ib], [jnp.full((16,), cc, jnp.int32),
                           jnp.full((16,), i, jnp.int32)])
            dm8 = jnp.bitwise_and(dsplat, 7)
            for j in range(8):
                rr[i, pl.ds(j * 16, 16)] = jnp.where(dm8 == j, den, zeros16)

    def scatter_chunk(ib, cc, rb):
        pltpu.async_copy(rows_l[rb], acc.at[dstb[ib].at[cc]], sem_sl[rb],
                         add=True)
        pltpu.async_copy(rows_r[rb], acc_den.at[d8b[ib].at[cc]], sem_sr[rb],
                         add=True)

    def wait_scatter(rb):
        pltpu.make_async_copy(rows_l[rb], acc.at[dstb[0].at[0]],
                              sem_sl[rb]).wait()
        pltpu.make_async_copy(rows_r[rb], acc_den.at[d8b[0].at[0]],
                              sem_sr[rb]).wait()

    issue_idx(0, 0)
    issue_idx(1, 1)
    wait_idx(0)
    fill_d8(0)
    issue_gather(0, 0, 0)

    @pl.loop(0, NSUPERS // 2)
    def _(spair):
        for sb in range(2):
            sup = 2 * spair + sb
            for cc in range(SUPER):
                rb = cc % 2
                wait_gather(rb)
                if cc < SUPER - 1:
                    if cc == 0 and sb == 0:
                        @pl.when(spair > 0)
                        def _():
                            wait_scatter(1 - rb)
                    elif cc == 0:
                        wait_scatter(1 - rb)
                    else:
                        wait_scatter(1 - rb)
                    issue_gather(sb, cc + 1, 1 - rb)
                elif sb == 0:
                    wait_idx(1)
                    fill_d8(1)
                    wait_scatter(1 - rb)
                    issue_gather(1, 0, 1 - rb)
                else:
                    @pl.when(spair < NSUPERS // 2 - 1)
                    def _():
                        wait_idx(0)
                        fill_d8(0)
                        wait_scatter(1 - rb)
                        issue_gather(0, 0, 1 - rb)
                compute_chunk(sb, cc, rb)
                scatter_chunk(sb, cc, rb)
            @pl.when(spair < NSUPERS // 2 - 1)
            def _():
                issue_idx(sup + 2, sb)

    wait_scatter(0)
    wait_scatter(1)

    plsc.subcore_barrier()

    pltpu.sync_copy(acc.at[pl.ds(s * ROWS_PER_TILE, ROWS_PER_TILE)],
                    out_msg_hbm.at[c_ax,
                                   pl.ds(s * ROWS_PER_TILE, ROWS_PER_TILE)])
    pltpu.sync_copy(
        acc_den.at[pl.ds(s * DROWS_PER_TILE, DROWS_PER_TILE)],
        out_den_hbm.at[c_ax, pl.ds(s * DROWS_PER_TILE, DROWS_PER_TILE)])


def _edge_pass(xl_pad, xr_pad, src_all, dst_all, att_flat):
    mesh = plsc.VectorSubcoreMesh(core_axis_name="c", subcore_axis_name="s")
    cp = pltpu.CompilerParams()
    if "needs_layout_passes" in pltpu.CompilerParams.__dataclass_fields__:
        cp = dataclasses.replace(cp, needs_layout_passes=False)
    run = pl.kernel(
        _edge_kernel,
        mesh=mesh,
        compiler_params=cp,
        out_type=[
            jax.ShapeDtypeStruct((2, NPAD, D), jnp.float32),
            jax.ShapeDtypeStruct((2, DROWS, D), jnp.float32),
        ],
        scratch_types=[
            pltpu.VMEM((SUPER, CHUNK), jnp.int32),
            pltpu.VMEM((SUPER, CHUNK), jnp.int32),
            pltpu.VMEM((SUPER, CHUNK), jnp.int32),
            pltpu.VMEM((SUPER, CHUNK), jnp.int32),
            pltpu.VMEM((SUPER, CHUNK), jnp.int32),
            pltpu.VMEM((SUPER, CHUNK), jnp.int32),
            pltpu.VMEM((CHUNK, D), jnp.float32),
            pltpu.VMEM((CHUNK, D), jnp.float32),
            pltpu.VMEM((CHUNK, D), jnp.float32),
            pltpu.VMEM((CHUNK, D), jnp.float32),
            pltpu.VMEM((D,), jnp.float32),
            pltpu.VMEM_SHARED((NPAD, D), jnp.float32),
            pltpu.VMEM_SHARED((DROWS, D), jnp.float32),
            pltpu.SemaphoreType.DMA,
            pltpu.SemaphoreType.DMA,
            pltpu.SemaphoreType.DMA,
            pltpu.SemaphoreType.DMA,
            pltpu.SemaphoreType.DMA,
            pltpu.SemaphoreType.DMA,
            pltpu.SemaphoreType.DMA,
            pltpu.SemaphoreType.DMA,
            pltpu.SemaphoreType.DMA,
            pltpu.SemaphoreType.DMA,
        ],
    )
    return run(xl_pad, xr_pad, src_all, dst_all, att_flat)


def _epi_body(num_ref, den_ref, x_ref, bias_ref, gamma_ref, beta_ref, o_ref):
    nv = num_ref[...]
    num = nv[0] + nv[1]
    dv = den_ref[...]
    den = dv[0] + dv[1]
    row = lax.broadcasted_iota(jnp.int32, (16, D), 0)
    col = lax.broadcasted_iota(jnp.int32, (16, D), 1)
    sel = jnp.where(col // C == row, 1.0, 0.0).astype(jnp.float32)
    denf = jax.lax.dot_general(den, sel, (((1,), (0,)), ((), ())),
                               preferred_element_type=jnp.float32)
    y = num / (denf + 1e-16) + bias_ref[...] + x_ref[...]
    mu = jnp.mean(y, axis=1, keepdims=True)
    d = y - mu
    var = jnp.mean(d * d, axis=1, keepdims=True)
    o_ref[...] = d * lax.rsqrt(var + 1e-5) * gamma_ref[...] + beta_ref[...]


def _epilogue(numer, den, x_pad, bias, gamma, beta):
    grid = NPAD // ROWB
    return pl.pallas_call(
        _epi_body,
        grid=(grid,),
        in_specs=[
            pl.BlockSpec((2, ROWB, D), lambda i: (0, i, 0)),
            pl.BlockSpec((2, ROWB, 16), lambda i: (0, i, 0)),
            pl.BlockSpec((ROWB, D), lambda i: (i, 0)),
            pl.BlockSpec((1, D), lambda i: (0, 0)),
            pl.BlockSpec((1, D), lambda i: (0, 0)),
            pl.BlockSpec((1, D), lambda i: (0, 0)),
        ],
        out_specs=pl.BlockSpec((ROWB, D), lambda i: (i, 0)),
        out_shape=jax.ShapeDtypeStruct((NPAD, D), jnp.float32),
    )(numer, den, x_pad, bias.reshape(1, D), gamma.reshape(1, D),
      beta.reshape(1, D))


def kernel(x, edge_index, W_l, b_l, W_r, b_r, att, bias, gamma, beta):
    x_pad = jnp.pad(x, ((0, NPAD - N), (0, 0)))
    loop = jnp.arange(N, dtype=jnp.int32)
    pad_idx = jnp.full((EPAD - E - N,), N, dtype=jnp.int32)
    src_all = jnp.concatenate([edge_index[0], loop, pad_idx])
    dst_all = jnp.concatenate([edge_index[1], loop, pad_idx])
    src_2d = src_all.reshape(EPAD // CHUNK, CHUNK)
    dst_2d = dst_all.reshape(EPAD // CHUNK, CHUNK)
    att_flat = att.reshape(H * C)

    xl_pad, xr_pad = _linear(x_pad, W_l, b_l, W_r, b_r)
    numer, den_parts = _edge_pass(xl_pad, xr_pad, src_2d, dst_2d, att_flat)
    den = den_parts.reshape(2, NPAD, 16)
    out = _epilogue(numer, den, x_pad, bias, gamma, beta)
    return out[:N]
